# baseline (speedup 1.0000x reference)
import functools

import jax
import jax.numpy as jnp
from jax import lax
from jax.experimental import pallas as pl
from jax.experimental.pallas import tpu as pltpu
from jax.experimental.pallas import tpu_sc as plsc

_N_U = 6000
_N_V = 4000
_N = _N_U + _N_V
_E = 320000
_D = 128
_H = _D // 2

_NC = 2
_NS = 16
_NW = _NC * _NS

_C = 125
_NP = 10240
_RPT = _NP // _NS
_HROWS = _E // _C
_IRPT = _HROWS // _NS
_PCH = 40

_mesh = plsc.VectorSubcoreMesh(
    core_axis_name="c", subcore_axis_name="s", num_cores=_NC, num_subcores=_NS
)


@functools.partial(
    pl.kernel,
    out_type=jax.ShapeDtypeStruct((2 * _NP, 16), jnp.float32),
    mesh=_mesh,
    scratch_types=[
        pltpu.VMEM((_IRPT, _C), jnp.int32),
        pltpu.VMEM((_C, 16), jnp.float32),
        pltpu.VMEM((_RPT, 16), jnp.float32),
        pltpu.VMEM_SHARED((_NP, 16), jnp.float32),
        pltpu.SemaphoreType.DMA,
    ],
    compiler_params=pltpu.CompilerParams(use_tc_tiling_on_sc=False),
)
def _degree_kernel(eidx_hbm, deg_hbm, idx_v, ones_v, zero_v, acc_sh, sem):
    c = lax.axis_index("c")
    s = lax.axis_index("s")

    @pl.loop(0, _C)
    def _(i):
        ones_v[i, :] = jnp.full((16,), 1.0, jnp.float32)

    @pl.loop(0, _RPT)
    def _(i):
        zero_v[i, :] = jnp.zeros((16,), jnp.float32)

    pltpu.sync_copy(zero_v, acc_sh.at[pl.ds(s * _RPT, _RPT)])
    plsc.subcore_barrier()

    pltpu.sync_copy(eidx_hbm.at[pl.ds(c * _HROWS + s * _IRPT, _IRPT)], idx_v)

    for j in range(8):
        pltpu.async_copy(ones_v, acc_sh.at[idx_v.at[j]], sem, add=True)

    @pl.loop(0, _IRPT - 8)
    def _(j):
        pltpu.make_async_copy(ones_v, acc_sh.at[idx_v.at[j]], sem).wait()
        pltpu.async_copy(ones_v, acc_sh.at[idx_v.at[j + 8]], sem, add=True)

    for j in range(_IRPT - 8, _IRPT):
        pltpu.make_async_copy(ones_v, acc_sh.at[idx_v.at[j]], sem).wait()

    plsc.subcore_barrier()
    pltpu.sync_copy(
        acc_sh.at[pl.ds(s * _RPT, _RPT)],
        deg_hbm.at[pl.ds(c * _NP + s * _RPT, _RPT)],
    )


@functools.partial(
    pl.kernel,
    out_type=jax.ShapeDtypeStruct((_NC * _NP, _H), jnp.float32),
    mesh=_mesh,
    scratch_types=[
        pltpu.VMEM((_PCH, _C), jnp.int32),
        pltpu.VMEM((_PCH, _C), jnp.int32),
        [pltpu.VMEM((_C, _H), jnp.float32) for _ in range(8)],
        pltpu.VMEM((128, _H), jnp.float32),
        pltpu.VMEM_SHARED((_NP, _H), jnp.float32),
        pltpu.SemaphoreType.DMA,
        pltpu.SemaphoreType.DMA,
        pltpu.SemaphoreType.DMA,
    ],
    compiler_params=pltpu.CompilerParams(use_tc_tiling_on_sc=False),
)
def _gather_scatter_kernel(
    tbl_hbm, eidx_hbm, out_hbm, sidx_v, didx_v, bufs, zero_v, acc_sh, gsa, gsb, ssem
):
    c = lax.axis_index("c")
    s = lax.axis_index("s")

    @pl.loop(0, 128)
    def _(i):
        for k in range(_H // 16):
            zero_v[i, pl.ds(k * 16, 16)] = jnp.zeros((16,), jnp.float32)

    for r in range(_RPT // 128):
        pltpu.sync_copy(zero_v, acc_sh.at[pl.ds(s * _RPT + r * 128, 128)])
    plsc.subcore_barrier()

    def fire_gathers(base, half, gsem):
        for b in range(4):
            pltpu.async_copy(tbl_hbm.at[sidx_v.at[base + b]], bufs[half * 4 + b], gsem)

    def drain_gathers(base, half, gsem):
        for b in range(4):
            pltpu.make_async_copy(
                tbl_hbm.at[sidx_v.at[base + b]], bufs[half * 4 + b], gsem
            ).wait()

    def fire_scatters(base, half):
        for b in range(4):
            pltpu.async_copy(
                bufs[half * 4 + b], acc_sh.at[didx_v.at[base + b]], ssem, add=True
            )

    def drain_scatters(base, half):
        for b in range(4):
            pltpu.make_async_copy(
                bufs[half * 4 + b], acc_sh.at[didx_v.at[base + b]], ssem
            ).wait()

    for p in range(_IRPT // _PCH):
        base = s * _IRPT + p * _PCH
        pltpu.sync_copy(eidx_hbm.at[pl.ds(c * _HROWS + base, _PCH)], sidx_v)
        pltpu.sync_copy(eidx_hbm.at[pl.ds(2 * _HROWS + base, _PCH)], didx_v)

        fire_gathers(0, 0, gsa)

        @pl.loop(0, _PCH // 8 - 1)
        def _(i):
            j = i * 8
            fire_gathers(j + 4, 1, gsb)
            drain_gathers(j, 0, gsa)
            fire_scatters(j, 0)
            drain_scatters(j, 0)
            fire_gathers(j + 8, 0, gsa)
            drain_gathers(j + 4, 1, gsb)
            fire_scatters(j + 4, 1)
            drain_scatters(j + 4, 1)

        j = _PCH - 8
        fire_gathers(j + 4, 1, gsb)
        drain_gathers(j, 0, gsa)
        fire_scatters(j, 0)
        drain_scatters(j, 0)
        drain_gathers(j + 4, 1, gsb)
        fire_scatters(j + 4, 1)
        drain_scatters(j + 4, 1)

    plsc.subcore_barrier()
    pltpu.sync_copy(
        acc_sh.at[pl.ds(s * _RPT, _RPT)],
        out_hbm.at[pl.ds(c * _NP + s * _RPT, _RPT)],
    )


_MBLK = 1000


def _mm_scale_body(u_ref, v_ref, uw_ref, vw_ref, d_ref, o_ref):
    i = pl.program_id(0)
    is_u = i < _N_U // _MBLK
    x = jnp.where(is_u, u_ref[...], v_ref[...])
    w = jnp.where(is_u, uw_ref[...], vw_ref[...])
    scale = lax.rsqrt(jnp.maximum(d_ref[:, 0:1], 1.0))
    r = jnp.dot(x, w, preferred_element_type=jnp.float32) * scale
    o_ref[0] = r[:, :_H]
    o_ref[1] = r[:, _H:]


def _mm_scale(u_f, v_f, u_w, v_w, deg):
    nu = _N_U // _MBLK
    return pl.pallas_call(
        _mm_scale_body,
        grid=(_N // _MBLK,),
        in_specs=[
            pl.BlockSpec((_MBLK, _D), lambda i: (jnp.minimum(i, _N_U // _MBLK - 1), 0)),
            pl.BlockSpec((_MBLK, _D), lambda i: (jnp.maximum(i - nu, 0), 0)),
            pl.BlockSpec((_D, _D), lambda i: (0, 0)),
            pl.BlockSpec((_D, _D), lambda i: (0, 0)),
            pl.BlockSpec((_MBLK, 16), lambda i: (i, 0)),
        ],
        out_specs=pl.BlockSpec((2, _MBLK, _H), lambda i: (0, i, 0)),
        out_shape=jax.ShapeDtypeStruct((2, _N, _H), jnp.float32),
    )(u_f, v_f, u_w, v_w, deg)


_FBLK = 80


def _final_body(pl_ref, pr_ref, d_ref, o_ref):
    scale = lax.rsqrt(jnp.maximum(d_ref[:, 0:1], 1.0))
    o_ref[...] = jnp.concatenate(
        [pl_ref[...] * scale, pr_ref[...] * scale], axis=1
    )


def _final(parts, deg):
    off = _NP // _FBLK
    return pl.pallas_call(
        _final_body,
        grid=(_N // _FBLK,),
        in_specs=[
            pl.BlockSpec((_FBLK, _H), lambda i: (i, 0)),
            pl.BlockSpec((_FBLK, _H), lambda i: (off + i, 0)),
            pl.BlockSpec((_FBLK, 16), lambda i: (off + i, 0)),
        ],
        out_specs=pl.BlockSpec((_FBLK, _D), lambda i: (i, 0)),
        out_shape=jax.ShapeDtypeStruct((_N, _D), jnp.float32),
    )(parts, parts, deg)


def kernel(u_f, v_f, edge_index, u_w, v_w):
    src2d = edge_index[0].reshape(_HROWS, _C)
    dst2d = edge_index[1].reshape(_HROWS, _C)
    eidx2d = jnp.concatenate([src2d, dst2d], axis=0)

    deg = _degree_kernel(eidx2d)

    tbl = _mm_scale(u_f, v_f, u_w, v_w, deg).reshape(2 * _N, _H)

    eidx_aug = jnp.concatenate([src2d, src2d + _N, dst2d], axis=0)

    parts = _gather_scatter_kernel(tbl, eidx_aug)
    return _final(parts, deg)


# speedup vs baseline: 1.3767x; 1.3767x over previous
import functools

import jax
import jax.numpy as jnp
from jax import lax
from jax.experimental import pallas as pl
from jax.experimental.pallas import tpu as pltpu
from jax.experimental.pallas import tpu_sc as plsc

_N_U = 6000
_N_V = 4000
_N = _N_U + _N_V
_E = 320000
_D = 128
_H = _D // 2

_NC = 2
_NS = 16
_NW = _NC * _NS

_C = 125
_NP = 10240
_RPT = _NP // _NS
_HROWS = _E // _C
_IRPT = _HROWS // _NS
_PCH = 40

_mesh = plsc.VectorSubcoreMesh(
    core_axis_name="c", subcore_axis_name="s", num_cores=_NC, num_subcores=_NS
)


@functools.partial(
    pl.kernel,
    out_type=[
        jax.ShapeDtypeStruct((_NP, 16), jnp.float32),
        jax.ShapeDtypeStruct((_NP, 16), jnp.float32),
    ],
    mesh=_mesh,
    scratch_types=[
        pltpu.VMEM((_IRPT, _C), jnp.int32),
        pltpu.VMEM((_C, 16), jnp.float32),
        pltpu.VMEM((_RPT, 16), jnp.float32),
        pltpu.VMEM_SHARED((_NP, 16), jnp.float32),
        pltpu.SemaphoreType.DMA,
    ],
    compiler_params=pltpu.CompilerParams(use_tc_tiling_on_sc=False),
)
def _degree_kernel(eidx_hbm, dout_hbm, din_hbm, idx_v, ones_v, zero_v, acc_sh, sem):
    c = lax.axis_index("c")
    s = lax.axis_index("s")

    @pl.loop(0, _C)
    def _(i):
        ones_v[i, :] = jnp.full((16,), 1.0, jnp.float32)

    @pl.loop(0, _RPT)
    def _(i):
        zero_v[i, :] = jnp.zeros((16,), jnp.float32)

    pltpu.sync_copy(zero_v, acc_sh.at[pl.ds(s * _RPT, _RPT)])
    plsc.subcore_barrier()

    pltpu.sync_copy(eidx_hbm.at[pl.ds(c * _HROWS + s * _IRPT, _IRPT)], idx_v)

    for j in range(8):
        pltpu.async_copy(ones_v, acc_sh.at[idx_v.at[j]], sem, add=True)

    @pl.loop(0, _IRPT - 8)
    def _(j):
        pltpu.make_async_copy(ones_v, acc_sh.at[idx_v.at[j]], sem).wait()
        pltpu.async_copy(ones_v, acc_sh.at[idx_v.at[j + 8]], sem, add=True)

    for j in range(_IRPT - 8, _IRPT):
        pltpu.make_async_copy(ones_v, acc_sh.at[idx_v.at[j]], sem).wait()

    plsc.subcore_barrier()

    @pl.when(c == 0)
    def _():
        pltpu.sync_copy(
            acc_sh.at[pl.ds(s * _RPT, _RPT)], dout_hbm.at[pl.ds(s * _RPT, _RPT)]
        )

    @pl.when(c == 1)
    def _():
        pltpu.sync_copy(
            acc_sh.at[pl.ds(s * _RPT, _RPT)], din_hbm.at[pl.ds(s * _RPT, _RPT)]
        )


@functools.partial(
    pl.kernel,
    out_type=[
        jax.ShapeDtypeStruct((_NP, _H), jnp.float32),
        jax.ShapeDtypeStruct((_NP, _H), jnp.float32),
    ],
    mesh=_mesh,
    scratch_types=[
        pltpu.VMEM((_PCH, _C), jnp.int32),
        pltpu.VMEM((_PCH, _C), jnp.int32),
        [pltpu.VMEM((_C, _H), jnp.float32) for _ in range(8)],
        pltpu.VMEM((128, _H), jnp.float32),
        pltpu.VMEM_SHARED((_NP, _H), jnp.float32),
        pltpu.SemaphoreType.DMA,
        pltpu.SemaphoreType.DMA,
        pltpu.SemaphoreType.DMA,
    ],
    compiler_params=pltpu.CompilerParams(use_tc_tiling_on_sc=False),
)
def _gather_scatter_kernel(
    tbl_hbm, eidx_hbm, out0_hbm, out1_hbm, sidx_v, didx_v, bufs, zero_v, acc_sh,
    gsa, gsb, ssem
):
    c = lax.axis_index("c")
    s = lax.axis_index("s")
    half_tbl = tbl_hbm.at[c]

    @pl.loop(0, 128)
    def _(i):
        for k in range(_H // 16):
            zero_v[i, pl.ds(k * 16, 16)] = jnp.zeros((16,), jnp.float32)

    for r in range(_RPT // 128):
        pltpu.sync_copy(zero_v, acc_sh.at[pl.ds(s * _RPT + r * 128, 128)])
    plsc.subcore_barrier()

    def fire_gathers(base, half, gsem):
        for b in range(4):
            pltpu.async_copy(half_tbl.at[sidx_v.at[base + b]], bufs[half * 4 + b], gsem)

    def drain_gathers(base, half, gsem):
        for b in range(4):
            pltpu.make_async_copy(
                half_tbl.at[sidx_v.at[base + b]], bufs[half * 4 + b], gsem
            ).wait()

    def fire_scatters(base, half):
        for b in range(4):
            pltpu.async_copy(
                bufs[half * 4 + b], acc_sh.at[didx_v.at[base + b]], ssem, add=True
            )

    def drain_scatters(base, half):
        for b in range(4):
            pltpu.make_async_copy(
                bufs[half * 4 + b], acc_sh.at[didx_v.at[base + b]], ssem
            ).wait()

    for p in range(_IRPT // _PCH):
        base = s * _IRPT + p * _PCH
        pltpu.sync_copy(eidx_hbm.at[pl.ds(base, _PCH)], sidx_v)
        pltpu.sync_copy(eidx_hbm.at[pl.ds(_HROWS + base, _PCH)], didx_v)

        fire_gathers(0, 0, gsa)

        @pl.loop(0, _PCH // 8 - 1)
        def _(i):
            j = i * 8
            fire_gathers(j + 4, 1, gsb)
            drain_gathers(j, 0, gsa)
            fire_scatters(j, 0)
            drain_scatters(j, 0)
            fire_gathers(j + 8, 0, gsa)
            drain_gathers(j + 4, 1, gsb)
            fire_scatters(j + 4, 1)
            drain_scatters(j + 4, 1)

        j = _PCH - 8
        fire_gathers(j + 4, 1, gsb)
        drain_gathers(j, 0, gsa)
        fire_scatters(j, 0)
        drain_scatters(j, 0)
        drain_gathers(j + 4, 1, gsb)
        fire_scatters(j + 4, 1)
        drain_scatters(j + 4, 1)

    plsc.subcore_barrier()

    @pl.when(c == 0)
    def _():
        pltpu.sync_copy(
            acc_sh.at[pl.ds(s * _RPT, _RPT)], out0_hbm.at[pl.ds(s * _RPT, _RPT)]
        )

    @pl.when(c == 1)
    def _():
        pltpu.sync_copy(
            acc_sh.at[pl.ds(s * _RPT, _RPT)], out1_hbm.at[pl.ds(s * _RPT, _RPT)]
        )


_MBLK = 1000


def _mm_scale_body(u_ref, v_ref, uw_ref, vw_ref, d_ref, o_ref):
    i = pl.program_id(0)
    is_u = i < _N_U // _MBLK
    x = jnp.where(is_u, u_ref[...], v_ref[...])
    w = jnp.where(is_u, uw_ref[...], vw_ref[...])
    scale = lax.rsqrt(jnp.maximum(d_ref[:, 0:1], 1.0))
    r = jnp.dot(x, w, preferred_element_type=jnp.float32) * scale
    o_ref[0] = r[:, :_H]
    o_ref[1] = r[:, _H:]


def _final_body(pl_ref, pr_ref, d_ref, o_ref):
    scale = lax.rsqrt(jnp.maximum(d_ref[:, 0:1], 1.0))
    o_ref[...] = jnp.concatenate(
        [pl_ref[...] * scale, pr_ref[...] * scale], axis=1
    )


def _final(p0, p1, deg_in):
    blk = 1000
    return pl.pallas_call(
        _final_body,
        grid=(_N // blk,),
        in_specs=[
            pl.BlockSpec((blk, _H), lambda i: (i, 0)),
            pl.BlockSpec((blk, _H), lambda i: (i, 0)),
            pl.BlockSpec((blk, 16), lambda i: (i, 0)),
        ],
        out_specs=pl.BlockSpec((blk, _D), lambda i: (i, 0)),
        out_shape=jax.ShapeDtypeStruct((_N, _D), jnp.float32),
    )(p0, p1, deg_in)


def _mm_scale(u_f, v_f, u_w, v_w, deg):
    nu = _N_U // _MBLK
    return pl.pallas_call(
        _mm_scale_body,
        grid=(_N // _MBLK,),
        in_specs=[
            pl.BlockSpec((_MBLK, _D), lambda i: (jnp.minimum(i, _N_U // _MBLK - 1), 0)),
            pl.BlockSpec((_MBLK, _D), lambda i: (jnp.maximum(i - nu, 0), 0)),
            pl.BlockSpec((_D, _D), lambda i: (0, 0)),
            pl.BlockSpec((_D, _D), lambda i: (0, 0)),
            pl.BlockSpec((_MBLK, 16), lambda i: (i, 0)),
        ],
        out_specs=pl.BlockSpec((2, _MBLK, _H), lambda i: (0, i, 0)),
        out_shape=jax.ShapeDtypeStruct((2, _N, _H), jnp.float32),
    )(u_f, v_f, u_w, v_w, deg)


def kernel(u_f, v_f, edge_index, u_w, v_w):
    eidx2d = edge_index.reshape(2 * _HROWS, _C)

    deg_out, deg_in = _degree_kernel(eidx2d)

    tbl = _mm_scale(u_f, v_f, u_w, v_w, deg_out)

    p0, p1 = _gather_scatter_kernel(tbl, eidx2d)
    return _final(p0, p1, deg_in)
